# baseline (speedup 1.0000x reference)
import functools

import jax
import jax.numpy as jnp
from jax.experimental import pallas as pl
from jax.experimental.pallas import tpu as pltpu

_INTERPRET = False


def _vec2d(a):
    return a.reshape(1, -1)


def _prep_rb(p):
    q = {
        'pre_g': _vec2d(p['pre_g']), 'pre_b': _vec2d(p['pre_b']),
        'lin1_WT': p['lin1_W'].T, 'lin1_b': _vec2d(p['lin1_b']),
        'n1_g': _vec2d(p['n1_g']), 'n1_b': _vec2d(p['n1_b']),
        'conv_W': p['conv_W'], 'conv_b': _vec2d(p['conv_b']),
        'n2_g': _vec2d(p['n2_g']), 'n2_b': _vec2d(p['n2_b']),
        'lin2_WT': p['lin2_W'].T, 'lin2_b': _vec2d(p['lin2_b']),
    }
    if 'skip_W' in p:
        q['skip_WT'] = p['skip_W'].T
        q['skip_b'] = _vec2d(p['skip_b'])
    return q


def _prep(params):
    return {
        'lin0_WT': params['lin0_W'].T, 'lin0_b': _vec2d(params['lin0_b']),
        'rb': [_prep_rb(p) for p in params['rb']],
        'shape_rb1': _prep_rb(params['shape_rb1']),
        'shape_rb2': _prep_rb(params['shape_rb2']),
        'shape_gn_g': _vec2d(params['shape_gn_g']),
        'shape_gn_b': _vec2d(params['shape_gn_b']),
        'shape_lin_WT': params['shape_lin_W'].T,
        'shape_lin_b': _vec2d(params['shape_lin_b']),
        'cam_gn_g': _vec2d(params['cam_gn_g']),
        'cam_gn_b': _vec2d(params['cam_gn_b']),
        'cam_glin_WT': params['cam_glin_W'].T,
        'cam_glin_b': _vec2d(params['cam_glin_b']),
        'cam_lin_WT': params['cam_lin_W'].T,
        'cam_lin_b': _vec2d(params['cam_lin_b']),
    }


_PREC = jax.lax.Precision.HIGHEST


def _mm(a, b):
    return jax.lax.dot_general(
        a, b, (((1,), (0,)), ((), ())), precision=_PREC,
        preferred_element_type=jnp.float32)


def _relu(a):
    return jnp.maximum(a, 0.0)


def _group_norm(y, g, b, eps=1e-5):
    n, c = y.shape
    ng = c // 8
    s = jnp.sum(y, axis=0, keepdims=True)
    ss = jnp.sum(y * y, axis=0, keepdims=True)
    ci = jax.lax.broadcasted_iota(jnp.int32, (c, ng), 0)
    gi = jax.lax.broadcasted_iota(jnp.int32, (c, ng), 1)
    sel = jnp.where(ci // 8 == gi, 1.0, 0.0)
    cnt = 8.0 * n
    mg = _mm(s, sel) / cnt
    vg = _mm(ss, sel) / cnt - mg * mg
    ig = jax.lax.rsqrt(vg + eps)
    back = (((1,), (1,)), ((), ()))
    mc = jax.lax.dot_general(mg, sel, back, precision=_PREC,
                             preferred_element_type=jnp.float32)
    ic = jax.lax.dot_general(ig, sel, back, precision=_PREC,
                             preferred_element_type=jnp.float32)
    return (y - mc) * ic * g + b


def _resblock(y, rp, adjm):
    t = _relu(_group_norm(y, rp['pre_g'][...], rp['pre_b'][...]))
    t = _mm(t, rp['lin1_WT'][...]) + rp['lin1_b'][...]
    t = _relu(_group_norm(t, rp['n1_g'][...], rp['n1_b'][...]))
    t = _mm(adjm, _mm(t, rp['conv_W'][...])) + rp['conv_b'][...]
    t = _relu(_group_norm(t, rp['n2_g'][...], rp['n2_b'][...]))
    t = _mm(t, rp['lin2_WT'][...]) + rp['lin2_b'][...]
    if 'skip_WT' in rp:
        y = _mm(y, rp['skip_WT'][...]) + rp['skip_b'][...]
    return y + t


def _gcn_body(treedef, n_w, *refs):
    x_ref, adj_ref = refs[0], refs[1]
    wrefs = refs[2:2 + n_w]
    shape_ref, cam_ref = refs[2 + n_w], refs[3 + n_w]
    p = jax.tree.unflatten(treedef, list(wrefs))
    adjm = adj_ref[...]

    xb = x_ref[0]
    h = jax.lax.dot_general(
        xb, p['lin0_WT'][...], (((0,), (0,)), ((), ())), precision=_PREC,
        preferred_element_type=jnp.float32) + p['lin0_b'][...]
    for rp in p['rb']:
        h = _resblock(h, rp, adjm)

    s = _resblock(h, p['shape_rb1'], adjm)
    s = _resblock(s, p['shape_rb2'], adjm)
    s = _relu(_group_norm(s, p['shape_gn_g'][...], p['shape_gn_b'][...]))
    so = _mm(s, p['shape_lin_WT'][...]) + p['shape_lin_b'][...]
    shape_ref[0] = so

    c = _relu(_group_norm(h, p['cam_gn_g'][...], p['cam_gn_b'][...]))
    c = _relu(_mm(c, p['cam_glin_WT'][...]) + p['cam_glin_b'][...])
    cam = jax.lax.dot_general(
        c, p['cam_lin_WT'][...], (((0,), (0,)), ((), ())), precision=_PREC,
        preferred_element_type=jnp.float32) + p['cam_lin_b'][...]
    cam_ref[0] = cam


def kernel(x, params, adj):
    bsz, cin0, nv = x.shape
    tp = _prep(params)
    leaves, treedef = jax.tree_util.tree_flatten(tp)
    n_w = len(leaves)

    in_specs = [
        pl.BlockSpec((1, cin0, nv), lambda b: (b, 0, 0)),
        pl.BlockSpec((nv, nv), lambda b: (0, 0)),
    ]
    for leaf in leaves:
        in_specs.append(pl.BlockSpec(leaf.shape, lambda b: (0, 0)))

    out_shapes = [
        jax.ShapeDtypeStruct((bsz, nv, 3), jnp.float32),
        jax.ShapeDtypeStruct((bsz, 1, 3), jnp.float32),
    ]
    out_specs = [
        pl.BlockSpec((1, nv, 3), lambda b: (b, 0, 0)),
        pl.BlockSpec((1, 1, 3), lambda b: (b, 0, 0)),
    ]

    shape_k, cam_k = pl.pallas_call(
        functools.partial(_gcn_body, treedef, n_w),
        grid=(bsz,),
        in_specs=in_specs,
        out_specs=out_specs,
        out_shape=out_shapes,
        compiler_params=pltpu.CompilerParams(
            dimension_semantics=("parallel",)),
        interpret=_INTERPRET,
    )(x, adj, *leaves)

    return jnp.swapaxes(shape_k, 1, 2), cam_k.reshape(bsz, 3)


# speedup vs baseline: 1.5144x; 1.5144x over previous
import functools

import jax
import jax.numpy as jnp
from jax.experimental import pallas as pl
from jax.experimental.pallas import tpu as pltpu

_INTERPRET = False
_NB = 2


def _vec2d(a):
    return a.reshape(1, -1)


def _split_w(a):
    hi = a.astype(jnp.bfloat16)
    lo = (a - hi.astype(jnp.float32)).astype(jnp.bfloat16)
    return {'hi': hi, 'lo': lo}


def _prep_rb(p):
    q = {
        'pre_g': _vec2d(p['pre_g']), 'pre_b': _vec2d(p['pre_b']),
        'lin1_WT': _split_w(p['lin1_W'].T), 'lin1_b': _vec2d(p['lin1_b']),
        'n1_g': _vec2d(p['n1_g']), 'n1_b': _vec2d(p['n1_b']),
        'conv_W': _split_w(p['conv_W']), 'conv_b': _vec2d(p['conv_b']),
        'n2_g': _vec2d(p['n2_g']), 'n2_b': _vec2d(p['n2_b']),
        'lin2_WT': _split_w(p['lin2_W'].T), 'lin2_b': _vec2d(p['lin2_b']),
    }
    if 'skip_W' in p:
        q['skip_WT'] = _split_w(p['skip_W'].T)
        q['skip_b'] = _vec2d(p['skip_b'])
    return q


def _prep(params):
    return {
        'lin0_WT': _split_w(params['lin0_W'].T),
        'lin0_b': _vec2d(params['lin0_b']),
        'rb': [_prep_rb(p) for p in params['rb']],
        'shape_rb1': _prep_rb(params['shape_rb1']),
        'shape_rb2': _prep_rb(params['shape_rb2']),
        'shape_gn_g': _vec2d(params['shape_gn_g']),
        'shape_gn_b': _vec2d(params['shape_gn_b']),
        'shape_lin_WT': _split_w(params['shape_lin_W'].T),
        'shape_lin_b': _vec2d(params['shape_lin_b']),
        'cam_gn_g': _vec2d(params['cam_gn_g']),
        'cam_gn_b': _vec2d(params['cam_gn_b']),
        'cam_glin_WT': _split_w(params['cam_glin_W'].T),
        'cam_glin_b': _vec2d(params['cam_glin_b']),
        'cam_lin_WT': _split_w(params['cam_lin_W'].T),
        'cam_lin_b': _vec2d(params['cam_lin_b']),
    }


_EXACT = jax.lax.Precision.HIGHEST


def _split_act(a):
    hi = a.astype(jnp.bfloat16)
    lo = (a - hi.astype(jnp.float32)).astype(jnp.bfloat16)
    return hi, lo


def _dot(a, b, dims):
    return jax.lax.dot_general(
        a, b, (dims, ((), ())), preferred_element_type=jnp.float32)


def _mm3(a, w, dims=((1,), (0,))):
    wh = w['hi'][...]
    wl = w['lo'][...]
    ah, al = _split_act(a)
    return _dot(ah, wh, dims) + (_dot(ah, wl, dims) + _dot(al, wh, dims))


def _adj_mm3(adjp, t):
    adjh, adjl = adjp
    th, tl = _split_act(t)
    dims = ((1,), (0,))
    return _dot(adjh, th, dims) + (_dot(adjh, tl, dims) + _dot(adjl, th, dims))


def _relu(a):
    return jnp.maximum(a, 0.0)


def _group_norm(y, g, b, eps=1e-5):
    n, c = y.shape
    ng = c // 8
    s = jnp.sum(y, axis=0, keepdims=True)
    ss = jnp.sum(y * y, axis=0, keepdims=True)
    stats = jnp.concatenate([s, ss], axis=0)
    ci = jax.lax.broadcasted_iota(jnp.int32, (c, ng), 0)
    gi = jax.lax.broadcasted_iota(jnp.int32, (c, ng), 1)
    sel = jnp.where(ci // 8 == gi, 1.0, 0.0)
    cnt = 8.0 * n
    gstats = jax.lax.dot_general(
        stats, sel, (((1,), (0,)), ((), ())), precision=_EXACT,
        preferred_element_type=jnp.float32) / cnt
    mg = gstats[0:1]
    vg = gstats[1:2] - mg * mg
    ig = jax.lax.rsqrt(vg + eps)
    mi = jnp.concatenate([mg, ig], axis=0)
    bc = jax.lax.dot_general(mi, sel, (((1,), (1,)), ((), ())),
                             precision=_EXACT,
                             preferred_element_type=jnp.float32)
    return (y - bc[0:1]) * bc[1:2] * g + b


def _resblock(y, rp, adjp):
    t = _relu(_group_norm(y, rp['pre_g'][...], rp['pre_b'][...]))
    t = _mm3(t, rp['lin1_WT']) + rp['lin1_b'][...]
    t = _relu(_group_norm(t, rp['n1_g'][...], rp['n1_b'][...]))
    t = _adj_mm3(adjp, _mm3(t, rp['conv_W'])) + rp['conv_b'][...]
    t = _relu(_group_norm(t, rp['n2_g'][...], rp['n2_b'][...]))
    t = _mm3(t, rp['lin2_WT']) + rp['lin2_b'][...]
    if 'skip_WT' in rp:
        y = _mm3(y, rp['skip_WT']) + rp['skip_b'][...]
    return y + t


def _one_element(xb, adjp, p):
    h = _mm3(xb, p['lin0_WT'], dims=((0,), (0,))) + p['lin0_b'][...]
    for rp in p['rb']:
        h = _resblock(h, rp, adjp)

    s = _resblock(h, p['shape_rb1'], adjp)
    s = _resblock(s, p['shape_rb2'], adjp)
    s = _relu(_group_norm(s, p['shape_gn_g'][...], p['shape_gn_b'][...]))
    so = _mm3(s, p['shape_lin_WT']) + p['shape_lin_b'][...]

    c = _relu(_group_norm(h, p['cam_gn_g'][...], p['cam_gn_b'][...]))
    c = _relu(_mm3(c, p['cam_glin_WT']) + p['cam_glin_b'][...])
    cam = _mm3(c, p['cam_lin_WT'], dims=((0,), (0,))) + p['cam_lin_b'][...]
    return so, cam


def _gcn_body(treedef, n_w, nb, *refs):
    x_ref, adjh_ref, adjl_ref = refs[0], refs[1], refs[2]
    wrefs = refs[3:3 + n_w]
    shape_ref, cam_ref = refs[3 + n_w], refs[4 + n_w]
    p = jax.tree.unflatten(treedef, list(wrefs))
    adjp = (adjh_ref[...], adjl_ref[...])
    for e in range(nb):
        so, cam = _one_element(x_ref[e], adjp, p)
        shape_ref[e] = so
        cam_ref[e] = cam


def kernel(x, params, adj):
    bsz, cin0, nv = x.shape
    nb = _NB if bsz % _NB == 0 else 1
    tp = _prep(params)
    adjs = _split_w(adj)
    leaves, treedef = jax.tree_util.tree_flatten(tp)
    n_w = len(leaves)

    in_specs = [
        pl.BlockSpec((nb, cin0, nv), lambda b: (b, 0, 0)),
        pl.BlockSpec((nv, nv), lambda b: (0, 0)),
        pl.BlockSpec((nv, nv), lambda b: (0, 0)),
    ]
    for leaf in leaves:
        in_specs.append(pl.BlockSpec(leaf.shape, lambda b: (0, 0)))

    out_shapes = [
        jax.ShapeDtypeStruct((bsz, nv, 3), jnp.float32),
        jax.ShapeDtypeStruct((bsz, 1, 3), jnp.float32),
    ]
    out_specs = [
        pl.BlockSpec((nb, nv, 3), lambda b: (b, 0, 0)),
        pl.BlockSpec((nb, 1, 3), lambda b: (b, 0, 0)),
    ]

    shape_k, cam_k = pl.pallas_call(
        functools.partial(_gcn_body, treedef, n_w, nb),
        grid=(bsz // nb,),
        in_specs=in_specs,
        out_specs=out_specs,
        out_shape=out_shapes,
        compiler_params=pltpu.CompilerParams(
            dimension_semantics=("parallel",)),
        interpret=_INTERPRET,
    )(x, adjs['hi'], adjs['lo'], *leaves)

    return jnp.swapaxes(shape_k, 1, 2), cam_k.reshape(bsz, 3)


# speedup vs baseline: 1.6985x; 1.1216x over previous
import functools

import jax
import jax.numpy as jnp
from jax.experimental import pallas as pl
from jax.experimental.pallas import tpu as pltpu

_INTERPRET = False
_NB = 2


def _vec2d(a):
    return a.reshape(1, -1)


def _bf16_pair(a):
    hi = a.astype(jnp.bfloat16)
    lo = (a - hi.astype(jnp.float32)).astype(jnp.bfloat16)
    return hi, lo


def _stack_w(a, axis=0):
    hi, lo = _bf16_pair(a)
    return jnp.concatenate([hi, lo, hi], axis=axis)


def _prep_rb(p):
    q = {
        'pre_g': _vec2d(p['pre_g']), 'pre_b': _vec2d(p['pre_b']),
        'lin1_W3': _stack_w(p['lin1_W'].T), 'lin1_b': _vec2d(p['lin1_b']),
        'n1_g': _vec2d(p['n1_g']), 'n1_b': _vec2d(p['n1_b']),
        'conv_W3': _stack_w(p['conv_W']), 'conv_b': _vec2d(p['conv_b']),
        'n2_g': _vec2d(p['n2_g']), 'n2_b': _vec2d(p['n2_b']),
        'lin2_W3': _stack_w(p['lin2_W'].T), 'lin2_b': _vec2d(p['lin2_b']),
    }
    if 'skip_W' in p:
        q['skip_W3'] = _stack_w(p['skip_W'].T)
        q['skip_b'] = _vec2d(p['skip_b'])
    return q


def _prep(params):
    return {
        'lin0_W3': _stack_w(params['lin0_W'].T),
        'lin0_b': _vec2d(params['lin0_b']),
        'rb': [_prep_rb(p) for p in params['rb']],
        'shape_rb1': _prep_rb(params['shape_rb1']),
        'shape_rb2': _prep_rb(params['shape_rb2']),
        'shape_gn_g': _vec2d(params['shape_gn_g']),
        'shape_gn_b': _vec2d(params['shape_gn_b']),
        'shape_lin_W3': _stack_w(params['shape_lin_W'].T),
        'shape_lin_b': _vec2d(params['shape_lin_b']),
        'cam_gn_g': _vec2d(params['cam_gn_g']),
        'cam_gn_b': _vec2d(params['cam_gn_b']),
        'cam_glin_W3': _stack_w(params['cam_glin_W'].T),
        'cam_glin_b': _vec2d(params['cam_glin_b']),
        'cam_lin_W3': _stack_w(params['cam_lin_W'].T),
        'cam_lin_b': _vec2d(params['cam_lin_b']),
    }


def _dot(a, b, dims=((1,), (0,))):
    return jax.lax.dot_general(
        a, b, (dims, ((), ())), preferred_element_type=jnp.float32)


def _mm3(a, w3, contract_lhs=1):
    ah, al = _bf16_pair(a)
    a3 = jnp.concatenate([ah, ah, al], axis=contract_lhs)
    return _dot(a3, w3[...], (((contract_lhs,), (0,))))


def _adj_mm3(adj3, t):
    th, tl = _bf16_pair(t)
    t3 = jnp.concatenate([th, tl, th], axis=0)
    return _dot(adj3, t3)


def _relu(a):
    return jnp.maximum(a, 0.0)


def _group_norm(y, g, b, eps=1e-5, relu=True):
    n, c = y.shape
    ng = c // 8
    s = jnp.sum(y, axis=0, keepdims=True)
    ss = jnp.sum(y * y, axis=0, keepdims=True)
    stats = jnp.concatenate([s, ss], axis=0)
    ci = jax.lax.broadcasted_iota(jnp.int32, (c, ng), 0)
    gi = jax.lax.broadcasted_iota(jnp.int32, (c, ng), 1)
    sel = jnp.where(ci // 8 == gi, 1.0, 0.0).astype(jnp.bfloat16)
    sth, stl = _bf16_pair(stats)
    st4 = jnp.concatenate([sth, stl], axis=0)
    g4 = _dot(st4, sel)
    cnt = 8.0 * n
    gstats = (g4[0:2] + g4[2:4]) / cnt
    mg = gstats[0:1]
    vg = gstats[1:2] - mg * mg
    ig = jax.lax.rsqrt(vg + eps)
    mi = jnp.concatenate([mg, ig], axis=0)
    mih, mil = _bf16_pair(mi)
    mi4 = jnp.concatenate([mih, mil], axis=0)
    bc4 = _dot(mi4, sel, (((1,), (1,))))
    mc = bc4[0:1] + bc4[2:3]
    ic = bc4[1:2] + bc4[3:4]
    scale = ic * g
    shift = b - mc * scale
    out = y * scale + shift
    return _relu(out) if relu else out


def _resblock(y, rp, adj3):
    t = _group_norm(y, rp['pre_g'][...], rp['pre_b'][...])
    t = _mm3(t, rp['lin1_W3']) + rp['lin1_b'][...]
    t = _group_norm(t, rp['n1_g'][...], rp['n1_b'][...])
    t = _adj_mm3(adj3, _mm3(t, rp['conv_W3'])) + rp['conv_b'][...]
    t = _group_norm(t, rp['n2_g'][...], rp['n2_b'][...])
    t = _mm3(t, rp['lin2_W3']) + rp['lin2_b'][...]
    if 'skip_W3' in rp:
        y = _mm3(y, rp['skip_W3']) + rp['skip_b'][...]
    return y + t


def _one_element(xb, adj3, p):
    h = _mm3(xb, p['lin0_W3'], contract_lhs=0) + p['lin0_b'][...]
    for rp in p['rb']:
        h = _resblock(h, rp, adj3)

    s = _resblock(h, p['shape_rb1'], adj3)
    s = _resblock(s, p['shape_rb2'], adj3)
    s = _group_norm(s, p['shape_gn_g'][...], p['shape_gn_b'][...])
    so = _mm3(s, p['shape_lin_W3']) + p['shape_lin_b'][...]

    c = _group_norm(h, p['cam_gn_g'][...], p['cam_gn_b'][...])
    c = _relu(_mm3(c, p['cam_glin_W3']) + p['cam_glin_b'][...])
    cam = _mm3(c, p['cam_lin_W3'], contract_lhs=0) + p['cam_lin_b'][...]
    return so, cam


def _gcn_body(treedef, n_w, nb, *refs):
    x_ref, adj3_ref = refs[0], refs[1]
    wrefs = refs[2:2 + n_w]
    shape_ref, cam_ref = refs[2 + n_w], refs[3 + n_w]
    p = jax.tree.unflatten(treedef, list(wrefs))
    adj3 = adj3_ref[...]
    for e in range(nb):
        so, cam = _one_element(x_ref[e], adj3, p)
        shape_ref[e] = so
        cam_ref[e] = cam


def kernel(x, params, adj):
    bsz, cin0, nv = x.shape
    nb = _NB if bsz % _NB == 0 else 1
    tp = _prep(params)
    adjh, adjl = _bf16_pair(adj)
    adj3 = jnp.concatenate([adjh, adjh, adjl], axis=1)
    leaves, treedef = jax.tree_util.tree_flatten(tp)
    n_w = len(leaves)

    in_specs = [
        pl.BlockSpec((nb, cin0, nv), lambda b: (b, 0, 0)),
        pl.BlockSpec(adj3.shape, lambda b: (0, 0)),
    ]
    for leaf in leaves:
        in_specs.append(pl.BlockSpec(leaf.shape, lambda b: (0, 0)))

    out_shapes = [
        jax.ShapeDtypeStruct((bsz, nv, 3), jnp.float32),
        jax.ShapeDtypeStruct((bsz, 1, 3), jnp.float32),
    ]
    out_specs = [
        pl.BlockSpec((nb, nv, 3), lambda b: (b, 0, 0)),
        pl.BlockSpec((nb, 1, 3), lambda b: (b, 0, 0)),
    ]

    shape_k, cam_k = pl.pallas_call(
        functools.partial(_gcn_body, treedef, n_w, nb),
        grid=(bsz // nb,),
        in_specs=in_specs,
        out_specs=out_specs,
        out_shape=out_shapes,
        compiler_params=pltpu.CompilerParams(
            dimension_semantics=("parallel",)),
        interpret=_INTERPRET,
    )(x, adj3, *leaves)

    return jnp.swapaxes(shape_k, 1, 2), cam_k.reshape(bsz, 3)
